# baseline (speedup 1.0000x reference)
import functools

import jax
import jax.numpy as jnp
from jax import lax
from jax.experimental import pallas as pl
from jax.experimental.pallas import tpu as pltpu
from jax.experimental.pallas import tpu_sc as plsc

_NV = 1024
_LD = 64
_B = 16
_HW = 32 * 32
_BB = 2
_M = _BB * _HW


def _argmin_rows(d):
    val = d
    idx = jax.lax.broadcasted_iota(jnp.int32, d.shape, 1)
    width = d.shape[1]
    while width > 128:
        half = width // 2
        vl, vr = val[:, :half], val[:, half:]
        il, ir = idx[:, :half], idx[:, half:]
        take = vr < vl
        val = jnp.where(take, vr, vl)
        idx = jnp.where(take, ir, il)
        width = half
    dmin = jnp.min(val, axis=1, keepdims=True)
    return jnp.min(jnp.where(val == dmin, idx, jnp.int32(_NV)),
                   axis=1, keepdims=True)


def _dist_body(z_ref, w_ref, idx_ref):
    zb = z_ref[...]
    zp = jnp.transpose(zb, (0, 2, 1)).reshape(_M, _LD)
    w = w_ref[...]

    zsq = jnp.sum(zp * zp, axis=1, keepdims=True)
    wt = w.T
    wsq = jnp.sum(wt * wt, axis=0, keepdims=True)
    mm2 = jax.lax.dot_general(zp, w + w, (((1,), (1,)), ((), ())),
                              preferred_element_type=jnp.float32)
    d = (zsq + wsq) - mm2

    idx_ref[0] = _argmin_rows(d).T


def _loss_body(z_ref, zq_ref, zqt_ref, loss_ref):
    zb = z_ref[...]
    zp = jnp.transpose(zb, (0, 2, 1))
    zq = zq_ref[:, :, :_LD]
    loss_ref[...] = (zq - zp) ** 2
    zqt_ref[...] = jnp.transpose(zq, (0, 2, 1))


def _sc_gather(table, idx):
    info = plsc.get_sparse_core_info()
    nw = info.num_cores * info.num_subcores
    b = idx.shape[0]
    d = table.shape[1]
    b_per_w = b // nw
    mesh = plsc.VectorSubcoreMesh(core_axis_name="c", subcore_axis_name="s")

    @functools.partial(
        pl.kernel, mesh=mesh,
        out_type=jax.ShapeDtypeStruct((b, d), jnp.float32),
        scratch_types=[
            pltpu.VMEM((b_per_w,), jnp.int32),
            pltpu.VMEM((b_per_w, d), jnp.float32),
            pltpu.SemaphoreType.DMA,
        ],
    )
    def k(table_hbm, idx_hbm, out_hbm, idx_v, rows_v, sem):
        wid = lax.axis_index("s") * info.num_cores + lax.axis_index("c")
        base = wid * b_per_w
        pltpu.sync_copy(idx_hbm.at[pl.ds(base, b_per_w)], idx_v)
        pltpu.async_copy(table_hbm.at[idx_v], rows_v, sem).wait()
        pltpu.sync_copy(rows_v, out_hbm.at[pl.ds(base, b_per_w)])

    return k(table, idx)


def kernel(z, W):
    z3 = z.reshape(_B, _LD, _HW)
    nsteps = _B // _BB

    idx3 = pl.pallas_call(
        _dist_body,
        grid=(nsteps,),
        in_specs=[
            pl.BlockSpec((_BB, _LD, _HW), lambda b: (b, 0, 0)),
            pl.BlockSpec((_NV, _LD), lambda b: (0, 0)),
        ],
        out_specs=pl.BlockSpec((1, 1, _M), lambda b: (b, 0, 0)),
        out_shape=jax.ShapeDtypeStruct((nsteps, 1, _M), jnp.int32),
    )(z3, W)
    min_encoding_indices = idx3.reshape(_B * _HW)

    w_pad = jnp.pad(W, ((0, 0), (0, 128 - _LD)))
    zq_flat = _sc_gather(w_pad, min_encoding_indices)

    zq3, loss3 = pl.pallas_call(
        _loss_body,
        grid=(nsteps,),
        in_specs=[
            pl.BlockSpec((_BB, _LD, _HW), lambda b: (b, 0, 0)),
            pl.BlockSpec((_BB, _HW, 128), lambda b: (b, 0, 0)),
        ],
        out_specs=[
            pl.BlockSpec((_BB, _LD, _HW), lambda b: (b, 0, 0)),
            pl.BlockSpec((_BB, _HW, _LD), lambda b: (b, 0, 0)),
        ],
        out_shape=[
            jax.ShapeDtypeStruct((_B, _LD, _HW), jnp.float32),
            jax.ShapeDtypeStruct((_B, _HW, _LD), jnp.float32),
        ],
    )(z3, zq_flat.reshape(_B, _HW, 128))

    z_q_out = zq3.reshape(_B, _LD, 32, 32)
    loss = loss3.reshape(_B, 32, 32, _LD)
    return (z_q_out, min_encoding_indices, loss)


# speedup vs baseline: 1.3022x; 1.3022x over previous
import jax
import jax.numpy as jnp
from jax.experimental import pallas as pl

_NV = 1024
_LD = 64
_B = 16
_HW = 32 * 32
_BB = 4
_M = _BB * _HW


def _argmin_rows(d):
    m = d.shape[0]
    val = d
    idx = jax.lax.broadcasted_iota(jnp.int32, d.shape, 1)
    width = d.shape[1]
    while width > 128:
        half = width // 2
        vl, vr = val[:, :half], val[:, half:]
        il, ir = idx[:, :half], idx[:, half:]
        take = vr < vl
        val = jnp.where(take, vr, vl)
        idx = jnp.where(take, ir, il)
        width = half
    dmin = jnp.min(val, axis=1, keepdims=True)
    return jnp.min(jnp.where(val == dmin, idx, jnp.int32(_NV)),
                   axis=1, keepdims=True)


def _vq_body(z_ref, w_ref, zq_ref, idx_ref, loss_ref):
    zb = z_ref[...]
    zp = jnp.transpose(zb, (0, 2, 1)).reshape(_M, _LD)
    w = w_ref[...]

    zsq = jnp.sum(zp * zp, axis=1, keepdims=True)
    wt = w.T
    wsq = jnp.sum(wt * wt, axis=0, keepdims=True)
    mm2 = jax.lax.dot_general(zp, w + w, (((1,), (1,)), ((), ())),
                              preferred_element_type=jnp.float32)
    d = (zsq + wsq) - mm2

    idxk = _argmin_rows(d)

    ids = jax.lax.broadcasted_iota(jnp.int32, d.shape, 1)
    oh = (ids == idxk).astype(jnp.bfloat16)
    zq = jax.lax.dot_general(oh, w.astype(jnp.bfloat16),
                             (((1,), (0,)), ((), ())),
                             preferred_element_type=jnp.float32)

    loss_ref[...] = ((zq - zp) ** 2).reshape(_BB, _HW, _LD)
    for i in range(_BB):
        zq_ref[i] = zq[i * _HW:(i + 1) * _HW].T
    idx_ref[0] = idxk.T


def kernel(z, W):
    z3 = z.reshape(_B, _LD, _HW)
    nsteps = _B // _BB
    zq3, idx3, loss3 = pl.pallas_call(
        _vq_body,
        grid=(nsteps,),
        in_specs=[
            pl.BlockSpec((_BB, _LD, _HW), lambda b: (b, 0, 0)),
            pl.BlockSpec((_NV, _LD), lambda b: (0, 0)),
        ],
        out_specs=[
            pl.BlockSpec((_BB, _LD, _HW), lambda b: (b, 0, 0)),
            pl.BlockSpec((1, 1, _M), lambda b: (b, 0, 0)),
            pl.BlockSpec((_BB, _HW, _LD), lambda b: (b, 0, 0)),
        ],
        out_shape=[
            jax.ShapeDtypeStruct((_B, _LD, _HW), jnp.float32),
            jax.ShapeDtypeStruct((nsteps, 1, _M), jnp.int32),
            jax.ShapeDtypeStruct((_B, _HW, _LD), jnp.float32),
        ],
    )(z3, W)
    z_q_out = zq3.reshape(_B, _LD, 32, 32)
    min_encoding_indices = idx3.reshape(_B * _HW)
    loss = loss3.reshape(_B, 32, 32, _LD)
    return (z_q_out, min_encoding_indices, loss)


# speedup vs baseline: 1.3044x; 1.0017x over previous
import jax
import jax.numpy as jnp
from jax.experimental import pallas as pl

_NV = 1024
_LD = 64
_B = 16
_HW = 32 * 32
_BB = 4
_M = _BB * _HW


def _argmin_rows(d):
    m = d.shape[0]
    val = d
    idx = jax.lax.broadcasted_iota(jnp.int32, d.shape, 1)
    width = d.shape[1]
    while width > 128:
        half = width // 2
        vl, vr = val[:, :half], val[:, half:]
        il, ir = idx[:, :half], idx[:, half:]
        take = vr < vl
        val = jnp.where(take, vr, vl)
        idx = jnp.where(take, ir, il)
        width = half
    dmin = jnp.min(val, axis=1, keepdims=True)
    return jnp.min(jnp.where(val == dmin, idx, jnp.int32(_NV)),
                   axis=1, keepdims=True)


def _vq_body(z_ref, w_ref, zq_ref, idx_ref, loss_ref):
    zb = z_ref[...]
    zp = jnp.transpose(zb, (0, 2, 1)).reshape(_M, _LD)
    w = w_ref[...]

    zsq = jnp.sum(zp * zp, axis=1, keepdims=True)
    wt = w.T
    wsq = jnp.sum(wt * wt, axis=0, keepdims=True)
    mm2 = jax.lax.dot_general(zp, w + w, (((1,), (1,)), ((), ())),
                              preferred_element_type=jnp.float32)
    d = (zsq + wsq) - mm2

    idxk = _argmin_rows(d)

    ids = jax.lax.broadcasted_iota(jnp.int32, d.shape, 1)
    oh = (ids == idxk).astype(jnp.bfloat16)
    zq = jax.lax.dot_general(oh, w.astype(jnp.bfloat16),
                             (((1,), (0,)), ((), ())),
                             preferred_element_type=jnp.float32)

    loss_ref[...] = ((zq - zp) ** 2).reshape(_BB, _HW, _LD)
    for i in range(_BB):
        zq_ref[i] = zq[i * _HW:(i + 1) * _HW].T
    idx_ref[...] = idxk.T.reshape(_M)


def kernel(z, W):
    z3 = z.reshape(_B, _LD, _HW)
    nsteps = _B // _BB
    zq3, idx3, loss3 = pl.pallas_call(
        _vq_body,
        grid=(nsteps,),
        in_specs=[
            pl.BlockSpec((_BB, _LD, _HW), lambda b: (b, 0, 0)),
            pl.BlockSpec((_NV, _LD), lambda b: (0, 0)),
        ],
        out_specs=[
            pl.BlockSpec((_BB, _LD, _HW), lambda b: (b, 0, 0)),
            pl.BlockSpec((_M,), lambda b: (b,)),
            pl.BlockSpec((_BB, _HW, _LD), lambda b: (b, 0, 0)),
        ],
        out_shape=[
            jax.ShapeDtypeStruct((_B, _LD, _HW), jnp.float32),
            jax.ShapeDtypeStruct((_B * _HW,), jnp.int32),
            jax.ShapeDtypeStruct((_B, _HW, _LD), jnp.float32),
        ],
    )(z3, W)
    z_q_out = zq3.reshape(_B, _LD, 32, 32)
    min_encoding_indices = idx3
    loss = loss3.reshape(_B, 32, 32, _LD)
    return (z_q_out, min_encoding_indices, loss)
